# baseline (speedup 1.0000x reference)
import functools

import jax
import jax.numpy as jnp
import numpy as np
from jax.experimental import pallas as pl
from jax.experimental.pallas import tpu as pltpu
from jax.experimental.pallas import tpu_sc as plsc

N, L, H, E = 1, 2048, 12, 64
C, ITER, BITS, TOPK = 128, 10, 32, 32
QT = 256
NEG_INF = np.float32(-np.inf)


def _dot(a, b, ca, cb):
    return jax.lax.dot_general(
        a, b, (((ca,), (cb,)), ((), ())),
        precision=jax.lax.Precision.HIGHEST,
        preferred_element_type=jnp.float32)


def _dotb(a, b, ca, cb):
    return jax.lax.dot_general(
        a.astype(jnp.bfloat16), b.astype(jnp.bfloat16),
        (((ca,), (cb,)), ((), ())),
        preferred_element_type=jnp.float32)


def _head_kernel(q_ref, k_ref, v_ref, pe_ref, pb_ref, init_ref,
                 out_ref, tab_ref, asn_ref):
    q = q_ref[0]
    k = k_ref[0]
    v = v_ref[0]
    pe = pe_ref[...]
    pb = pb_ref[...]
    init_row = init_ref[...]
    temp = np.float32(1.0 / np.sqrt(E))

    proj = _dotb(q, pe, 1, 1) + pb
    hbits = (proj > 0).astype(jnp.float32)

    iota_l = jax.lax.broadcasted_iota(jnp.int32, (L, C), 0)
    sel_T = (iota_l == init_row).astype(jnp.float32)
    cent = _dotb(sel_T, hbits, 0, 0)

    iota_c0 = jax.lax.broadcasted_iota(jnp.int32, (C, L), 0)
    iota_cf = iota_c0.astype(jnp.float32)
    assign_row = jnp.zeros((1, L), jnp.int32)
    onehot_T = jnp.zeros((C, L), jnp.float32)
    counts = jnp.zeros((C, 1), jnp.float32)
    for _ in range(ITER):
        csum = jnp.sum(cent, axis=1, keepdims=True)
        bias = csum * 2048.0 + (65536.0 + iota_cf)
        key = bias - 4096.0 * _dotb(cent, hbits, 1, 1)
        kmin = jnp.min(key, axis=0, keepdims=True)
        assign_row = jnp.bitwise_and(kmin.astype(jnp.int32), 2047)
        onehot_T = (iota_c0 == assign_row).astype(jnp.float32)
        counts = jnp.sum(onehot_T, axis=1, keepdims=True)
        sums = _dotb(onehot_T, hbits, 1, 0)
        new_cent = (sums / jnp.maximum(counts, 1.0) > 0.5).astype(jnp.float32)
        cent = jnp.where(counts > 0, new_cent, cent)

    q_grouped = _dotb(onehot_T, q, 1, 0) / jnp.maximum(counts, 1.0)
    qk = _dotb(temp * q_grouped, k, 1, 1)

    iota_s = jax.lax.broadcasted_iota(jnp.int32, (C, L), 1)
    u = jax.lax.bitcast_convert_type(qk, jnp.int32)
    skey = jnp.where(u < 0, u ^ jnp.int32(0x7FFFFFFF), u)
    lo = jnp.full((C, 1), jnp.iinfo(jnp.int32).min, jnp.int32)
    hi = jnp.full((C, 1), jnp.iinfo(jnp.int32).max, jnp.int32)
    for _ in range(32):
        mid = (lo & hi) + ((lo ^ hi) >> 1)
        cnt = jnp.sum((skey > mid).astype(jnp.int32), axis=1, keepdims=True)
        pred = cnt >= TOPK
        lo = jnp.where(pred, mid, lo)
        hi = jnp.where(pred, hi, mid)
    ge = skey > lo
    strict = skey > lo + 1
    topmask = strict.astype(jnp.float32)
    eq0 = jnp.where(ge, 1.0, 0.0) - topmask
    m0 = jnp.sum(topmask, axis=1, keepdims=True).astype(jnp.int32)
    need0 = jnp.where(counts > 0, TOPK - m0, 0)

    def _tie_cond(state):
        _, _, need = state
        return jnp.any(need > 0)

    def _tie_body(state):
        tm, eq, need = state
        first = jnp.min(jnp.where(eq > 0, iota_s, L), axis=1, keepdims=True)
        hit = jnp.logical_and(iota_s == first, need > 0)
        tm = jnp.where(hit, 1.0, tm)
        eq = jnp.where(hit, 0.0, eq)
        need = need - (need > 0).astype(jnp.int32)
        return tm, eq, need

    topmask, _, _ = jax.lax.while_loop(
        _tie_cond, _tie_body, (topmask, eq0, need0))

    e_full = jnp.exp(qk)
    z = jnp.sum(e_full, axis=1, keepdims=True)
    e_b = e_full * (1.0 - topmask)
    a_bottomk = jnp.sum(e_b, axis=1, keepdims=True) / z
    v_bottom_c = _dotb(e_b, v, 1, 0) / z

    tab_ref[0] = jnp.concatenate(
        [v_bottom_c, jnp.broadcast_to(a_bottomk, (C, 16)),
         jnp.zeros((C, 128 - E - 16), jnp.float32)], axis=1)
    asn_ref[0] = assign_row + pl.program_id(0) * C

    qs = temp * q
    for t in range(L // QT):
        sl = slice(t * QT, (t + 1) * QT)
        oh_t = onehot_T[:, sl]
        mask_t = _dotb(oh_t, topmask, 0, 0)
        s_t = _dotb(qs[sl, :], k, 1, 1)
        e_t = jnp.exp(jnp.where(mask_t > 0, s_t, NEG_INF))
        zinv_t = 1.0 / jnp.sum(e_t, axis=1, keepdims=True)
        out_ref[0, sl, :] = _dotb(e_t, v, 1, 0) * zinv_t


def _make_sc_combine(rows, e):
    info = plsc.get_sparse_core_info()
    nw = info.num_cores * info.num_subcores
    nsub = 2
    chunk = rows // (nw * nsub)
    mesh = plsc.VectorSubcoreMesh(core_axis_name="c", subcore_axis_name="s")

    @functools.partial(
        pl.kernel, mesh=mesh,
        out_type=jax.ShapeDtypeStruct((rows, e), jnp.float32),
        scratch_types=[
            pltpu.VMEM((chunk,), jnp.int32),
            pltpu.VMEM((chunk, 128), jnp.float32),
            pltpu.VMEM((chunk, e), jnp.float32),
            pltpu.SemaphoreType.DMA,
        ],
    )
    def sc_combine(vtop_hbm, tab_hbm, asn_hbm, out_hbm,
                   idx_v, rows_v, vtop_v, sem):
        wid = jax.lax.axis_index("s") * info.num_cores + jax.lax.axis_index("c")

        for s in range(nsub):
            base = (wid * nsub + s) * chunk
            pltpu.sync_copy(asn_hbm.at[pl.ds(base, chunk)], idx_v)
            pltpu.async_copy(tab_hbm.at[idx_v], rows_v, sem).wait()
            pltpu.sync_copy(vtop_hbm.at[pl.ds(base, chunk)], vtop_v)

            def row(r, carry):
                w = 1.0 - rows_v[r, pl.ds(e, 16)]
                for j in range(e // 16):
                    vtop_v[r, pl.ds(j * 16, 16)] = (
                        rows_v[r, pl.ds(j * 16, 16)]
                        + w * vtop_v[r, pl.ds(j * 16, 16)])
                return carry

            jax.lax.fori_loop(0, chunk, row, 0)
            pltpu.sync_copy(vtop_v, out_hbm.at[pl.ds(base, chunk)])

    return sc_combine


def kernel(queries, keys, values, planes):
    n, l, h, e = queries.shape
    q = jnp.transpose(queries, (0, 2, 1, 3)).reshape(h, l, e)
    k = jnp.transpose(keys, (0, 2, 1, 3)).reshape(h, l, e)
    v = jnp.transpose(values, (0, 2, 1, 3)).reshape(h, l, e)
    pe = planes[:, :e]
    pb = planes[:, e].reshape(1, BITS)
    init_idx = jnp.linspace(0, l - 1, C).astype(jnp.int32).reshape(1, C)

    hspec = pl.BlockSpec((1, l, e), lambda i: (i, 0, 0))
    full = lambda s: pl.BlockSpec(s, lambda i: tuple(0 for _ in s))

    def tc_half(qh, kh, vh, hh):
        return pl.pallas_call(
            _head_kernel,
            grid=(hh,),
            in_specs=[hspec, hspec, hspec,
                      full((BITS, e)), full((1, BITS)), full((1, C))],
            out_specs=[hspec,
                       pl.BlockSpec((1, C, 128), lambda i: (i, 0, 0)),
                       pl.BlockSpec((1, 1, l), lambda i: (i, 0, 0))],
            out_shape=[jax.ShapeDtypeStruct((hh, l, e), jnp.float32),
                       jax.ShapeDtypeStruct((hh, C, 128), jnp.float32),
                       jax.ShapeDtypeStruct((hh, 1, l), jnp.int32)],
            compiler_params=pltpu.CompilerParams(
                dimension_semantics=("parallel",)),
        )(qh, kh, vh, pe, pb, init_idx)

    hh = h // 2
    sc_combine = _make_sc_combine(hh * l, e)
    outs = []
    for half in range(2):
        sl = slice(half * hh, (half + 1) * hh)
        vtop, tab, asn = tc_half(q[sl], k[sl], v[sl], hh)
        outs.append(sc_combine(vtop.reshape(hh * l, e),
                               tab.reshape(hh * C, 128),
                               asn.reshape(hh * l)))
    out = jnp.concatenate(outs, axis=0)
    return jnp.transpose(out.reshape(n, h, l, e), (0, 2, 1, 3))


# speedup vs baseline: 1.1581x; 1.1581x over previous
import functools

import jax
import jax.numpy as jnp
import numpy as np
from jax.experimental import pallas as pl
from jax.experimental.pallas import tpu as pltpu
from jax.experimental.pallas import tpu_sc as plsc

N, L, H, E = 1, 2048, 12, 64
C, ITER, BITS, TOPK = 128, 10, 32, 32
QT = 256
NEG_INF = np.float32(-np.inf)


def _dot(a, b, ca, cb):
    return jax.lax.dot_general(
        a, b, (((ca,), (cb,)), ((), ())),
        precision=jax.lax.Precision.HIGHEST,
        preferred_element_type=jnp.float32)


def _dotb(a, b, ca, cb):
    return jax.lax.dot_general(
        a.astype(jnp.bfloat16), b.astype(jnp.bfloat16),
        (((ca,), (cb,)), ((), ())),
        preferred_element_type=jnp.float32)


def _head_kernel(q_ref, k_ref, v_ref, pe_ref, pb_ref, init_ref,
                 out_ref, tab_ref, asn_ref):
    q = q_ref[0]
    k = k_ref[0]
    v = v_ref[0]
    pe = pe_ref[...]
    pb = pb_ref[...]
    init_row = init_ref[...]
    temp = np.float32(1.0 / np.sqrt(E))

    proj = _dotb(q, pe, 1, 1) + pb
    hbits = (proj > 0).astype(jnp.float32)

    iota_l = jax.lax.broadcasted_iota(jnp.int32, (L, C), 0)
    sel_T = (iota_l == init_row).astype(jnp.float32)
    cent = _dotb(sel_T, hbits, 0, 0)

    iota_c0 = jax.lax.broadcasted_iota(jnp.int32, (C, L), 0)
    iota_cf = iota_c0.astype(jnp.float32)
    assign_row = jnp.zeros((1, L), jnp.int32)
    onehot_T = jnp.zeros((C, L), jnp.float32)
    counts = jnp.zeros((C, 1), jnp.float32)
    for _ in range(ITER):
        csum = jnp.sum(cent, axis=1, keepdims=True)
        bias = csum * 2048.0 + (65536.0 + iota_cf)
        key = bias - 4096.0 * _dotb(cent, hbits, 1, 1)
        kmin = jnp.min(key, axis=0, keepdims=True)
        assign_row = jnp.bitwise_and(kmin.astype(jnp.int32), 2047)
        onehot_T = (iota_c0 == assign_row).astype(jnp.float32)
        counts = jnp.sum(onehot_T, axis=1, keepdims=True)
        sums = _dotb(onehot_T, hbits, 1, 0)
        new_cent = (sums / jnp.maximum(counts, 1.0) > 0.5).astype(jnp.float32)
        cent = jnp.where(counts > 0, new_cent, cent)

    q_grouped = _dotb(onehot_T, q, 1, 0) / jnp.maximum(counts, 1.0)
    qk = _dotb(temp * q_grouped, k, 1, 1)

    iota_s = jax.lax.broadcasted_iota(jnp.int32, (C, L), 1)
    u = jax.lax.bitcast_convert_type(qk, jnp.int32)
    skey = jnp.where(u < 0, u ^ jnp.int32(0x7FFFFFFF), u)
    lo = jnp.full((C, 1), jnp.iinfo(jnp.int32).min, jnp.int32)
    hi = jnp.full((C, 1), jnp.iinfo(jnp.int32).max, jnp.int32)
    for _ in range(32):
        mid = (lo & hi) + ((lo ^ hi) >> 1)
        cnt = jnp.sum((skey > mid).astype(jnp.int32), axis=1, keepdims=True)
        pred = cnt >= TOPK
        lo = jnp.where(pred, mid, lo)
        hi = jnp.where(pred, hi, mid)
    ge = skey > lo
    strict = skey > lo + 1
    topmask = strict.astype(jnp.float32)
    eq0 = jnp.where(ge, 1.0, 0.0) - topmask
    m0 = jnp.sum(topmask, axis=1, keepdims=True).astype(jnp.int32)
    need0 = jnp.where(counts > 0, TOPK - m0, 0)

    def _tie_cond(state):
        _, _, need = state
        return jnp.any(need > 0)

    def _tie_body(state):
        tm, eq, need = state
        first = jnp.min(jnp.where(eq > 0, iota_s, L), axis=1, keepdims=True)
        hit = jnp.logical_and(iota_s == first, need > 0)
        tm = jnp.where(hit, 1.0, tm)
        eq = jnp.where(hit, 0.0, eq)
        need = need - (need > 0).astype(jnp.int32)
        return tm, eq, need

    topmask, _, _ = jax.lax.while_loop(
        _tie_cond, _tie_body, (topmask, eq0, need0))

    e_full = jnp.exp(qk)
    z = jnp.sum(e_full, axis=1, keepdims=True)
    e_b = e_full * (1.0 - topmask)
    a_bottomk = jnp.sum(e_b, axis=1, keepdims=True) / z
    v_bottom_c = _dotb(e_b, v, 1, 0) / z

    tab_ref[0] = jnp.concatenate(
        [v_bottom_c, jnp.broadcast_to(a_bottomk, (C, 16)),
         jnp.zeros((C, 128 - E - 16), jnp.float32)], axis=1)
    asn_ref[0] = assign_row + pl.program_id(0) * C

    qs = temp * q
    for t in range(L // QT):
        sl = slice(t * QT, (t + 1) * QT)
        oh_t = onehot_T[:, sl]
        mask_t = _dotb(oh_t, topmask, 0, 0)
        s_t = _dotb(qs[sl, :], k, 1, 1)
        e_t = jnp.exp(jnp.where(mask_t > 0, s_t, NEG_INF))
        zinv_t = 1.0 / jnp.sum(e_t, axis=1, keepdims=True)
        out_ref[0, sl, :] = _dotb(e_t, v, 1, 0) * zinv_t


def _make_sc_combine(rows, e):
    info = plsc.get_sparse_core_info()
    nw = info.num_cores * info.num_subcores
    nsub = 2
    chunk = rows // (nw * nsub)
    mesh = plsc.VectorSubcoreMesh(core_axis_name="c", subcore_axis_name="s")

    @functools.partial(
        pl.kernel, mesh=mesh,
        out_type=jax.ShapeDtypeStruct((rows, e), jnp.float32),
        scratch_types=[
            pltpu.VMEM((chunk,), jnp.int32),
            pltpu.VMEM((chunk, 128), jnp.float32),
            pltpu.VMEM((chunk, e), jnp.float32),
            pltpu.SemaphoreType.DMA,
        ],
    )
    def sc_combine(vtop_hbm, tab_hbm, asn_hbm, out_hbm,
                   idx_v, rows_v, vtop_v, sem):
        wid = jax.lax.axis_index("s") * info.num_cores + jax.lax.axis_index("c")

        for s in range(nsub):
            base = (wid * nsub + s) * chunk
            pltpu.sync_copy(asn_hbm.at[pl.ds(base, chunk)], idx_v)
            pltpu.async_copy(tab_hbm.at[idx_v], rows_v, sem).wait()
            pltpu.sync_copy(vtop_hbm.at[pl.ds(base, chunk)], vtop_v)

            def row(r, carry):
                w = 1.0 - rows_v[r, pl.ds(e, 16)]
                for j in range(e // 16):
                    vtop_v[r, pl.ds(j * 16, 16)] = (
                        rows_v[r, pl.ds(j * 16, 16)]
                        + w * vtop_v[r, pl.ds(j * 16, 16)])
                return carry

            jax.lax.fori_loop(0, chunk, row, 0)
            pltpu.sync_copy(vtop_v, out_hbm.at[pl.ds(base, chunk)])

    return sc_combine


def kernel(queries, keys, values, planes):
    n, l, h, e = queries.shape
    q = jnp.transpose(queries, (0, 2, 1, 3)).reshape(h, l, e)
    k = jnp.transpose(keys, (0, 2, 1, 3)).reshape(h, l, e)
    v = jnp.transpose(values, (0, 2, 1, 3)).reshape(h, l, e)
    pe = planes[:, :e]
    pb = planes[:, e].reshape(1, BITS)
    init_idx = jnp.linspace(0, l - 1, C).astype(jnp.int32).reshape(1, C)

    hspec = pl.BlockSpec((1, l, e), lambda i: (i, 0, 0))
    full = lambda s: pl.BlockSpec(s, lambda i: tuple(0 for _ in s))
    vtop, tab, asn = pl.pallas_call(
        _head_kernel,
        grid=(h,),
        in_specs=[hspec, hspec, hspec,
                  full((BITS, e)), full((1, BITS)), full((1, C))],
        out_specs=[hspec,
                   pl.BlockSpec((1, C, 128), lambda i: (i, 0, 0)),
                   pl.BlockSpec((1, 1, l), lambda i: (i, 0, 0))],
        out_shape=[jax.ShapeDtypeStruct((h, l, e), jnp.float32),
                   jax.ShapeDtypeStruct((h, C, 128), jnp.float32),
                   jax.ShapeDtypeStruct((h, 1, l), jnp.int32)],
        compiler_params=pltpu.CompilerParams(
            dimension_semantics=("parallel",)),
    )(q, k, v, pe, pb, init_idx)

    sc_combine = _make_sc_combine(h * l, e)
    out = sc_combine(vtop.reshape(h * l, e),
                     tab.reshape(h * C, 128),
                     asn.reshape(h * l))
    return jnp.transpose(out.reshape(n, h, l, e), (0, 2, 1, 3))
